# baseline (speedup 1.0000x reference)
import functools

import jax
import jax.numpy as jnp
from jax import lax
from jax.experimental import pallas as pl
from jax.experimental.pallas import tpu as pltpu
from jax.experimental.pallas import tpu_sc as plsc

F32 = jnp.float32

_N = 50000
_E = 800000
_C = 5000
_A = 1000

NPAD = 50176
EPAD = 819200
CPAD = 8192
APAD = 1024
BE = 512
W = 40

_MESH = dict(core_axis_name="c", subcore_axis_name="s", num_cores=2,
             num_subcores=16)
_SC_PARAMS = pltpu.CompilerParams(use_tc_tiling_on_sc=False)


_CPT = EPAD // 32 // 128


def _gather_body(fea_hbm, lnp_hbm, sidx_hbm, nidx_hbm, p_out, l_out,
                 sidx_v, nidx_v, cbuf0, cbuf1, cbuf2, cbuf3,
                 lbuf0, lbuf1, sem0, sem1):
    c = lax.axis_index("c")
    s = lax.axis_index("s")
    wid = s * 2 + c
    base = wid * _CPT
    cbuf = ((cbuf0, cbuf1), (cbuf2, cbuf3))
    lbuf = (lbuf0, lbuf1)
    sems = (sem0, sem1)

    pltpu.sync_copy(sidx_hbm.at[pl.ds(base, _CPT)], sidx_v)
    pltpu.sync_copy(nidx_hbm.at[pl.ds(base, _CPT)], nidx_v)

    def fire(gi, p):
        pltpu.async_copy(fea_hbm.at[sidx_v.at[gi]], cbuf[p][0], sems[p])
        pltpu.async_copy(fea_hbm.at[nidx_v.at[gi]], cbuf[p][1], sems[p])
        pltpu.async_copy(lnp_hbm.at[nidx_v.at[gi]], lbuf[p], sems[p])

    def drain_write(gi, p):
        e0 = (base + gi) * 128
        for half in (0, 1):
            pltpu.make_async_copy(
                p_out.at[pl.ds(0, 128), pl.ds(64 * half, 64)],
                cbuf[p][half], sems[p]).wait()
            pltpu.sync_copy(cbuf[p][half],
                            p_out.at[pl.ds(e0, 128), pl.ds(64 * half, 64)])
        pltpu.make_async_copy(l_out.at[pl.ds(0, 128)], lbuf[p],
                              sems[p]).wait()
        pltpu.sync_copy(lbuf[p], l_out.at[pl.ds(e0, 128)])

    fire(0, 0)

    def body(di, carry):
        for p in (0, 1):
            gi = 2 * di + p

            @pl.when(gi + 1 < _CPT)
            def _():
                fire(gi + 1, 1 - p)

            drain_write(gi, p)
        return carry

    lax.fori_loop(0, _CPT // 2, body, 0)


def _sc_gather(fea, lnp, sidx2d, nidx2d):
    mesh = plsc.VectorSubcoreMesh(**_MESH)
    bf16 = jnp.bfloat16
    fn = pl.kernel(
        _gather_body,
        out_type=[jax.ShapeDtypeStruct((EPAD, 128), bf16),
                  jax.ShapeDtypeStruct((EPAD, 16), F32)],
        mesh=mesh,
        scratch_types=[pltpu.VMEM((_CPT, 128), jnp.int32),
                       pltpu.VMEM((_CPT, 128), jnp.int32),
                       pltpu.VMEM((128, 64), bf16),
                       pltpu.VMEM((128, 64), bf16),
                       pltpu.VMEM((128, 64), bf16),
                       pltpu.VMEM((128, 64), bf16),
                       pltpu.VMEM((128, 16), F32),
                       pltpu.VMEM((128, 16), F32),
                       pltpu.SemaphoreType.DMA,
                       pltpu.SemaphoreType.DMA],
        compiler_params=_SC_PARAMS,
    )
    return fn(fea, lnp, sidx2d, nidx2d)


def _make_segsum_body(nrows, nacc, ch):
    cpt = nrows // 16 // ch
    rpt = nacc // 16

    def body(zab_hbm, idx_hbm, zeros_hbm, outa_hbm, outb_hbm,
             idx0, idx1, vbuf0, vbuf1, sem0, sem1, acc):
        c = lax.axis_index("c")
        s = lax.axis_index("s")
        r0 = s * rpt
        pltpu.sync_copy(zeros_hbm, acc.at[pl.ds(r0, rpt)])
        plsc.subcore_barrier()
        idx = (idx0, idx1)
        vbuf = (vbuf0, vbuf1)
        sems = (sem0, sem1)

        def scan(col):
            def fire(ci, p):
                ch0 = s * cpt + ci
                pltpu.async_copy(idx_hbm.at[pl.ds(ch0, 1)], idx[p], sems[p])
                pltpu.async_copy(
                    zab_hbm.at[pl.ds(ch0 * ch, ch), pl.ds(col, W)],
                    vbuf[p], sems[p])

            def drain_scatter(p):
                pltpu.make_async_copy(idx_hbm.at[pl.ds(0, 1)], idx[p],
                                      sems[p]).wait()
                pltpu.make_async_copy(
                    zab_hbm.at[pl.ds(0, ch), pl.ds(col, W)], vbuf[p],
                    sems[p]).wait()
                pltpu.sync_copy(vbuf[p], acc.at[idx[p].at[0]], add=True)

            fire(0, 0)

            def chunk(di, carry):
                for p in (0, 1):
                    ci = 2 * di + p

                    @pl.when(ci + 1 < cpt)
                    def _():
                        fire(ci + 1, 1 - p)

                    drain_scatter(p)
                return carry
            lax.fori_loop(0, cpt // 2, chunk, 0)
            if cpt % 2 == 1:
                drain_scatter(0)

        @pl.when(c == 0)
        def _():
            scan(0)

        @pl.when(c == 1)
        def _():
            scan(48)

        plsc.subcore_barrier()

        @pl.when(c == 0)
        def _():
            pltpu.sync_copy(acc.at[pl.ds(r0, rpt)],
                            outa_hbm.at[pl.ds(r0, rpt)])

        @pl.when(c == 1)
        def _():
            pltpu.sync_copy(acc.at[pl.ds(r0, rpt)],
                            outb_hbm.at[pl.ds(r0, rpt)])

    return body, rpt


def _sc_segsum(zab, idx2d, nrows, nacc, ch):
    body, rpt = _make_segsum_body(nrows, nacc, ch)
    mesh = plsc.VectorSubcoreMesh(**_MESH)
    zeros = jnp.zeros((rpt, W), F32)
    fn = pl.kernel(
        body,
        out_type=[jax.ShapeDtypeStruct((nacc, W), F32),
                  jax.ShapeDtypeStruct((nacc, W), F32)],
        mesh=mesh,
        scratch_types=[pltpu.VMEM((1, ch), jnp.int32),
                       pltpu.VMEM((1, ch), jnp.int32),
                       pltpu.VMEM((ch, W), F32),
                       pltpu.VMEM((ch, W), F32),
                       pltpu.SemaphoreType.DMA,
                       pltpu.SemaphoreType.DMA,
                       pltpu.VMEM_SHARED((nacc, W), F32)],
        compiler_params=_SC_PARAMS,
    )
    return fn(zab, idx2d, zeros)


def _embed_body(ef_ref, sf_ref, w_ref, wea, bea, wsa, wsb, bsa, pow_ref,
                fea_ref, feab_ref, lnp_ref, lnw_ref):
    ef = jnp.dot(ef_ref[...], wea[...], preferred_element_type=F32) + bea[...]
    sf = (jnp.dot(sf_ref[...], wsa[...], preferred_element_type=F32)
          + w_ref[...] * wsb[...] + bsa[...])
    fea = jnp.concatenate([ef, sf], axis=1)
    fea_ref[...] = fea
    feab_ref[...] = fea.astype(jnp.bfloat16)
    lnw = jnp.log(w_ref[...])
    lnw_ref[...] = lnw
    lnp_ref[...] = jnp.broadcast_to(pow_ref[0, 0] * lnw, (lnw.shape[0], 16))


def _tc_embed(efe, sfe, ew, wea, bea, wsa, wsb, bsa, pow1):
    grid = (NPAD // BE,)
    return pl.pallas_call(
        _embed_body,
        grid=grid,
        in_specs=[
            pl.BlockSpec((BE, 128), lambda i: (i, 0)),
            pl.BlockSpec((BE, 128), lambda i: (i, 0)),
            pl.BlockSpec((BE, 1), lambda i: (i, 0)),
            pl.BlockSpec((128, 32), lambda i: (0, 0)),
            pl.BlockSpec((1, 32), lambda i: (0, 0)),
            pl.BlockSpec((128, 32), lambda i: (0, 0)),
            pl.BlockSpec((1, 32), lambda i: (0, 0)),
            pl.BlockSpec((1, 32), lambda i: (0, 0)),
            pl.BlockSpec(memory_space=pltpu.SMEM),
        ],
        out_specs=[
            pl.BlockSpec((BE, 64), lambda i: (i, 0)),
            pl.BlockSpec((BE, 64), lambda i: (i, 0)),
            pl.BlockSpec((BE, 16), lambda i: (i, 0)),
            pl.BlockSpec((BE, 1), lambda i: (i, 0)),
        ],
        out_shape=[jax.ShapeDtypeStruct((NPAD, 64), F32),
                   jax.ShapeDtypeStruct((NPAD, 64), jnp.bfloat16),
                   jax.ShapeDtypeStruct((NPAD, 16), F32),
                   jax.ShapeDtypeStruct((NPAD, 1), F32)],
    )(efe, sfe, ew, wea, bea, wsa, wsb, bsa, pow1)


def _edge_body(p_ref, l_ref, a_ref, c_ref, b1_ref, w2g_ref, b2g_ref,
               w2m_ref, b2m_ref, zab_ref):
    i = pl.program_id(0)
    bf16 = jnp.bfloat16
    pt = p_ref[...]
    sfe = pt[:, :64].astype(bf16)
    bf = pt[:, 64:].astype(bf16)
    row8 = lax.broadcasted_iota(jnp.int32, (BE, BE // 8), 0) // 8
    col8 = lax.broadcasted_iota(jnp.int32, (BE, BE // 8), 1)
    asel = jnp.where(row8 == col8, 1.0, 0.0)
    rep = jnp.dot(asel, l_ref[...], preferred_element_type=F32)
    lane = lax.broadcasted_iota(jnp.int32, (BE, 128), 1)
    rmod = (lax.broadcasted_iota(jnp.int32, (BE, 128), 0) % 8) * 16
    lnpw = jnp.sum(jnp.where(lane == rmod, rep, 0.0), axis=1, keepdims=True)
    h = (jnp.dot(sfe, a_ref[...].astype(bf16), preferred_element_type=F32)
         + jnp.dot(bf, c_ref[...].astype(bf16),
                   preferred_element_type=F32) + b1_ref[...])
    h = jnp.where(h > 0, h, 0.01 * h)
    hg = h[:, :256]
    hm = h[:, 256:]
    g = jnp.sum(hg * w2g_ref[...], axis=1, keepdims=True) + b2g_ref[0, 0]
    eid = i * BE + lax.broadcasted_iota(jnp.int32, (BE, 1), 0)
    u = jnp.where(eid < _E, jnp.exp(g + lnpw), 0.0)
    msg = jnp.dot(hm.astype(bf16), w2m_ref[...].astype(bf16),
                  preferred_element_type=F32) + b2m_ref[...]
    z = msg * u
    zab_ref[...] = jnp.concatenate(
        [z[:, :32], u, jnp.zeros((BE, 15), F32), z[:, 32:],
         jnp.zeros((BE, 48), F32)], axis=1)


def _tc_edge(pg, lg, wp):
    grid = (EPAD // BE,)
    return pl.pallas_call(
        _edge_body,
        grid=grid,
        in_specs=[
            pl.BlockSpec((BE, 128), lambda i: (i, 0)),
            pl.BlockSpec((BE // 8, 128), lambda i: (i, 0)),
            pl.BlockSpec((64, 512), lambda i: (0, 0)),
            pl.BlockSpec((64, 512), lambda i: (0, 0)),
            pl.BlockSpec((1, 512), lambda i: (0, 0)),
            pl.BlockSpec((1, 256), lambda i: (0, 0)),
            pl.BlockSpec(memory_space=pltpu.SMEM),
            pl.BlockSpec((256, 64), lambda i: (0, 0)),
            pl.BlockSpec((1, 64), lambda i: (0, 0)),
        ],
        out_specs=pl.BlockSpec((BE, 128), lambda i: (i, 0)),
        out_shape=jax.ShapeDtypeStruct((EPAD, 128), F32),
    )(pg, lg, wp["a"], wp["c"], wp["b1"], wp["w2g"], wp["b2g"], wp["w2m"],
      wp["b2m"])


def _finalize_body(sa_ref, sb_ref, fp_ref, lnw_ref, pow_ref,
                   fea_ref, feab_ref, lnp_ref):
    sa = sa_ref[...]
    sb = sb_ref[...]
    z = jnp.concatenate([sa[:, :32], sb[:, :32]], axis=1)
    u = sa[:, 32:33]
    fea = z / (u + 1e-10) + fp_ref[...]
    fea_ref[...] = fea
    feab_ref[...] = fea.astype(jnp.bfloat16)
    lnpw = pow_ref[0, 0] * lnw_ref[...]
    lnp_ref[...] = jnp.broadcast_to(lnpw, (lnpw.shape[0], 16))


def _tc_finalize(sa, sb, fprev, lnw, pownext):
    grid = (NPAD // BE,)
    return pl.pallas_call(
        _finalize_body,
        grid=grid,
        in_specs=[
            pl.BlockSpec((BE, W), lambda i: (i, 0)),
            pl.BlockSpec((BE, W), lambda i: (i, 0)),
            pl.BlockSpec((BE, 64), lambda i: (i, 0)),
            pl.BlockSpec((BE, 1), lambda i: (i, 0)),
            pl.BlockSpec(memory_space=pltpu.SMEM),
        ],
        out_specs=[pl.BlockSpec((BE, 64), lambda i: (i, 0)),
                   pl.BlockSpec((BE, 64), lambda i: (i, 0)),
                   pl.BlockSpec((BE, 16), lambda i: (i, 0))],
        out_shape=[jax.ShapeDtypeStruct((NPAD, 64), F32),
                   jax.ShapeDtypeStruct((NPAD, 64), jnp.bfloat16),
                   jax.ShapeDtypeStruct((NPAD, 16), F32)],
    )(sa, sb, fprev, lnw, pownext)


def _cry_body(f_ref, lnw_ref, a_ref, b1_ref, w2g_ref, b2g_ref, w2m_ref,
              b2m_ref, pow_ref, zab_ref):
    i = pl.program_id(0)
    bf16 = jnp.bfloat16
    x = f_ref[...]
    h = jnp.dot(x.astype(bf16), a_ref[...].astype(bf16),
                preferred_element_type=F32) + b1_ref[...]
    h = jnp.where(h > 0, h, 0.01 * h)
    hg = h[:, :256]
    hm = h[:, 256:]
    g = jnp.sum(hg * w2g_ref[...], axis=1, keepdims=True) + b2g_ref[0, 0]
    nid = i * BE + lax.broadcasted_iota(jnp.int32, (BE, 1), 0)
    u = jnp.where(nid < _N, jnp.exp(g + pow_ref[0, 0] * lnw_ref[...]), 0.0)
    msg = jnp.dot(hm.astype(bf16), w2m_ref[...].astype(bf16),
                  preferred_element_type=F32) + b2m_ref[...]
    z = msg * u
    zab_ref[...] = jnp.concatenate(
        [z[:, :32], u, jnp.zeros((BE, 15), F32), z[:, 32:],
         jnp.zeros((BE, 48), F32)], axis=1)


def _tc_cry(fea, lnw, wp):
    grid = (NPAD // BE,)
    return pl.pallas_call(
        _cry_body,
        grid=grid,
        in_specs=[
            pl.BlockSpec((BE, 64), lambda i: (i, 0)),
            pl.BlockSpec((BE, 1), lambda i: (i, 0)),
            pl.BlockSpec((64, 512), lambda i: (0, 0)),
            pl.BlockSpec((1, 512), lambda i: (0, 0)),
            pl.BlockSpec((1, 256), lambda i: (0, 0)),
            pl.BlockSpec(memory_space=pltpu.SMEM),
            pl.BlockSpec((256, 64), lambda i: (0, 0)),
            pl.BlockSpec((1, 64), lambda i: (0, 0)),
            pl.BlockSpec(memory_space=pltpu.SMEM),
        ],
        out_specs=pl.BlockSpec((BE, 128), lambda i: (i, 0)),
        out_shape=jax.ShapeDtypeStruct((NPAD, 128), F32),
    )(fea, lnw, wp["a"], wp["b1"], wp["w2g"], wp["b2g"], wp["w2m"],
      wp["b2m"], wp["pow"])


def _cryfin_body(sa_ref, sb_ref, zab_ref):
    i = pl.program_id(0)
    sa = sa_ref[...]
    sb = sb_ref[...]
    z = jnp.concatenate([sa[:, :32], sb[:, :32]], axis=1)
    u = sa[:, 32:33]
    cry = z / (u + 1e-10)
    rid = i * BE + lax.broadcasted_iota(jnp.int32, (BE, 1), 0)
    ones = jnp.where(rid < _C, 1.0, 0.0)
    zab_ref[...] = jnp.concatenate(
        [cry[:, :32], ones, jnp.zeros((BE, 15), F32), cry[:, 32:],
         jnp.zeros((BE, 48), F32)], axis=1)


def _tc_cryfin(sa, sb):
    grid = (CPAD // BE,)
    return pl.pallas_call(
        _cryfin_body,
        grid=grid,
        in_specs=[pl.BlockSpec((BE, W), lambda i: (i, 0)),
                  pl.BlockSpec((BE, W), lambda i: (i, 0))],
        out_specs=pl.BlockSpec((BE, 128), lambda i: (i, 0)),
        out_shape=jax.ShapeDtypeStruct((CPAD, 128), F32),
    )(sa, sb)


def _final_body(sa_ref, sb_ref, out_ref):
    sa = sa_ref[...]
    sb = sb_ref[...]
    sums = jnp.concatenate([sa[:, :32], sb[:, :32]], axis=1)
    counts = sa[:, 32:33]
    out_ref[...] = sums / jnp.maximum(counts, 1.0)


def _tc_final(sa, sb):
    grid = (APAD // BE,)
    return pl.pallas_call(
        _final_body,
        grid=grid,
        in_specs=[pl.BlockSpec((BE, W), lambda i: (i, 0)),
                  pl.BlockSpec((BE, W), lambda i: (i, 0))],
        out_specs=pl.BlockSpec((BE, 64), lambda i: (i, 0)),
        out_shape=jax.ShapeDtypeStruct((APAD, 64), F32),
    )(sa, sb)


def _pack_wap(p, din):
    wg1, bg1 = p["gate"]["fcs"][0]
    wg2, bg2 = p["gate"]["out"]
    wm1, bm1 = p["msg"]["fcs"][0]
    wm2, bm2 = p["msg"]["out"]
    d = {}
    if din == 128:
        d["a"] = jnp.concatenate([wg1[:, :64].T, wm1[:, :64].T], axis=1)
        d["c"] = jnp.concatenate([wg1[:, 64:].T, wm1[:, 64:].T], axis=1)
    else:
        d["a"] = jnp.concatenate([wg1.T, wm1.T], axis=1)
    d["b1"] = jnp.concatenate([bg1, bm1]).reshape(1, 512)
    d["w2g"] = wg2.reshape(1, 256)
    d["b2g"] = bg2.reshape(1, 1)
    d["w2m"] = wm2.T
    d["b2m"] = bm2.reshape(1, 64)
    d["pow"] = p["pow"].reshape(1, 1)
    return d


def kernel(elem_weights, elem_fea, sym_fea, self_fea_idx, nbr_fea_idx,
           cry_elem_idx, aug_cry_idx, params):
    ew = jnp.pad(elem_weights, ((0, NPAD - _N), (0, 0)), constant_values=1.0)
    efe = jnp.pad(elem_fea, ((0, NPAD - _N), (0, 0)))
    sfe = jnp.pad(sym_fea, ((0, NPAD - _N), (0, 0)))
    sidx = jnp.pad(self_fea_idx, (0, EPAD - _E)).reshape(EPAD // 128, 128)
    nidx = jnp.pad(nbr_fea_idx, (0, EPAD - _E)).reshape(EPAD // 128, 128)
    sidx64 = sidx.reshape(EPAD // 64, 64)
    cidx = jnp.pad(cry_elem_idx, (0, NPAD - _N)).reshape(NPAD // 64, 64)
    aidx = jnp.pad(aug_cry_idx, (0, CPAD - _C)).reshape(CPAD // 128, 128)

    graph_w = [_pack_wap(gp["heads"][0], 128) for gp in params["graphs"]]
    cry_w = _pack_wap(params["cry_pool"][0], 64)

    wemb, bemb = params["elem_embed"]
    wsym, bsym = params["sym_embed"]
    fea, feab, lnp, lnw = _tc_embed(
        efe, sfe, ew, wemb.T, bemb.reshape(1, 32), wsym[:, :128].T,
        wsym[:, 128:].T, bsym.reshape(1, 32), graph_w[0]["pow"])

    for layer in range(3):
        pg, lg = _sc_gather(feab, lnp, sidx, nidx)
        zab = _tc_edge(pg, lg.reshape(EPAD // 8, 128), graph_w[layer])
        sa, sb = _sc_segsum(zab, sidx64, EPAD, NPAD, 64)
        pnext = (graph_w[layer + 1]["pow"] if layer < 2
                 else graph_w[layer]["pow"])
        fea, feab, lnp = _tc_finalize(sa, sb, fea, lnw, pnext)

    zab = _tc_cry(fea, lnw, cry_w)
    sa, sb = _sc_segsum(zab, cidx, NPAD, CPAD, 64)
    zab2 = _tc_cryfin(sa, sb)
    sa2, sb2 = _sc_segsum(zab2, aidx, CPAD, APAD, 128)
    out = _tc_final(sa2, sb2)
    return out[:_A]


# speedup vs baseline: 1.2333x; 1.2333x over previous
import functools

import jax
import jax.numpy as jnp
from jax import lax
from jax.experimental import pallas as pl
from jax.experimental.pallas import tpu as pltpu
from jax.experimental.pallas import tpu_sc as plsc

F32 = jnp.float32

_N = 50000
_E = 800000
_C = 5000
_A = 1000

NPAD = 50176
EPAD = 819200
CPAD = 8192
APAD = 1024
BE = 1024
W = 40

_MESH = dict(core_axis_name="c", subcore_axis_name="s", num_cores=2,
             num_subcores=16)
_SC_PARAMS = pltpu.CompilerParams(use_tc_tiling_on_sc=False)


_CPT = EPAD // 32 // 128


def _gather_body(fea_hbm, lnp_hbm, sidx_hbm, nidx_hbm, p_out, l_out,
                 sidx_v, nidx_v, cbuf0, cbuf1, cbuf2, cbuf3,
                 lbuf0, lbuf1, sem0, sem1):
    c = lax.axis_index("c")
    s = lax.axis_index("s")
    wid = s * 2 + c
    base = wid * _CPT
    cbuf = ((cbuf0, cbuf1), (cbuf2, cbuf3))
    lbuf = (lbuf0, lbuf1)
    sems = (sem0, sem1)

    pltpu.sync_copy(sidx_hbm.at[pl.ds(base, _CPT)], sidx_v)
    pltpu.sync_copy(nidx_hbm.at[pl.ds(base, _CPT)], nidx_v)

    def fire(gi, p):
        pltpu.async_copy(fea_hbm.at[sidx_v.at[gi]], cbuf[p][0], sems[p])
        pltpu.async_copy(fea_hbm.at[nidx_v.at[gi]], cbuf[p][1], sems[p])
        pltpu.async_copy(lnp_hbm.at[nidx_v.at[gi]], lbuf[p], sems[p])

    def drain_write(gi, p):
        e0 = (base + gi) * 128
        for half in (0, 1):
            pltpu.make_async_copy(
                p_out.at[pl.ds(0, 128), pl.ds(64 * half, 64)],
                cbuf[p][half], sems[p]).wait()
            pltpu.sync_copy(cbuf[p][half],
                            p_out.at[pl.ds(e0, 128), pl.ds(64 * half, 64)])
        pltpu.make_async_copy(l_out.at[pl.ds(0, 128)], lbuf[p],
                              sems[p]).wait()
        pltpu.sync_copy(lbuf[p], l_out.at[pl.ds(e0, 128)])

    fire(0, 0)

    def body(di, carry):
        for p in (0, 1):
            gi = 2 * di + p

            @pl.when(gi + 1 < _CPT)
            def _():
                fire(gi + 1, 1 - p)

            drain_write(gi, p)
        return carry

    lax.fori_loop(0, _CPT // 2, body, 0)


def _sc_gather(fea, lnp, sidx2d, nidx2d):
    mesh = plsc.VectorSubcoreMesh(**_MESH)
    bf16 = jnp.bfloat16
    fn = pl.kernel(
        _gather_body,
        out_type=[jax.ShapeDtypeStruct((EPAD, 128), F32),
                  jax.ShapeDtypeStruct((EPAD, 16), F32)],
        mesh=mesh,
        scratch_types=[pltpu.VMEM((_CPT, 128), jnp.int32),
                       pltpu.VMEM((_CPT, 128), jnp.int32),
                       pltpu.VMEM((128, 64), F32),
                       pltpu.VMEM((128, 64), F32),
                       pltpu.VMEM((128, 64), F32),
                       pltpu.VMEM((128, 64), F32),
                       pltpu.VMEM((128, 16), F32),
                       pltpu.VMEM((128, 16), F32),
                       pltpu.SemaphoreType.DMA,
                       pltpu.SemaphoreType.DMA],
        compiler_params=_SC_PARAMS,
    )
    return fn(fea, lnp, sidx2d, nidx2d)


def _make_segsum_body(nrows, nacc, ch):
    cpt = nrows // 16 // ch
    rpt = nacc // 16

    def body(zab_hbm, idx_hbm, zeros_hbm, outa_hbm, outb_hbm,
             idx0, idx1, vbuf0, vbuf1, sem0, sem1, acc):
        c = lax.axis_index("c")
        s = lax.axis_index("s")
        r0 = s * rpt
        pltpu.sync_copy(zeros_hbm, acc.at[pl.ds(r0, rpt)])
        plsc.subcore_barrier()
        idx = (idx0, idx1)
        vbuf = (vbuf0, vbuf1)
        sems = (sem0, sem1)

        def scan(col):
            def fire(ci, p):
                ch0 = s * cpt + ci
                pltpu.async_copy(idx_hbm.at[pl.ds(ch0, 1)], idx[p], sems[p])
                pltpu.async_copy(
                    zab_hbm.at[pl.ds(ch0 * ch, ch), pl.ds(col, W)],
                    vbuf[p], sems[p])

            def drain_scatter(p):
                pltpu.make_async_copy(idx_hbm.at[pl.ds(0, 1)], idx[p],
                                      sems[p]).wait()
                pltpu.make_async_copy(
                    zab_hbm.at[pl.ds(0, ch), pl.ds(col, W)], vbuf[p],
                    sems[p]).wait()
                pltpu.sync_copy(vbuf[p], acc.at[idx[p].at[0]], add=True)

            fire(0, 0)

            def chunk(di, carry):
                for p in (0, 1):
                    ci = 2 * di + p

                    @pl.when(ci + 1 < cpt)
                    def _():
                        fire(ci + 1, 1 - p)

                    drain_scatter(p)
                return carry
            lax.fori_loop(0, cpt // 2, chunk, 0)
            if cpt % 2 == 1:
                drain_scatter(0)

        @pl.when(c == 0)
        def _():
            scan(0)

        @pl.when(c == 1)
        def _():
            scan(48)

        plsc.subcore_barrier()

        @pl.when(c == 0)
        def _():
            pltpu.sync_copy(acc.at[pl.ds(r0, rpt)],
                            outa_hbm.at[pl.ds(r0, rpt)])

        @pl.when(c == 1)
        def _():
            pltpu.sync_copy(acc.at[pl.ds(r0, rpt)],
                            outb_hbm.at[pl.ds(r0, rpt)])

    return body, rpt


def _sc_segsum(zab, idx2d, nrows, nacc, ch):
    body, rpt = _make_segsum_body(nrows, nacc, ch)
    mesh = plsc.VectorSubcoreMesh(**_MESH)
    zeros = jnp.zeros((rpt, W), F32)
    fn = pl.kernel(
        body,
        out_type=[jax.ShapeDtypeStruct((nacc, W), F32),
                  jax.ShapeDtypeStruct((nacc, W), F32)],
        mesh=mesh,
        scratch_types=[pltpu.VMEM((1, ch), jnp.int32),
                       pltpu.VMEM((1, ch), jnp.int32),
                       pltpu.VMEM((ch, W), F32),
                       pltpu.VMEM((ch, W), F32),
                       pltpu.SemaphoreType.DMA,
                       pltpu.SemaphoreType.DMA,
                       pltpu.VMEM_SHARED((nacc, W), F32)],
        compiler_params=_SC_PARAMS,
    )
    return fn(zab, idx2d, zeros)


def _embed_body(ef_ref, sf_ref, w_ref, wea, bea, wsa, wsb, bsa, pow_ref,
                fea_ref, lnp_ref, lnw_ref):
    ef = jnp.dot(ef_ref[...], wea[...], preferred_element_type=F32) + bea[...]
    sf = (jnp.dot(sf_ref[...], wsa[...], preferred_element_type=F32)
          + w_ref[...] * wsb[...] + bsa[...])
    fea_ref[...] = jnp.concatenate([ef, sf], axis=1)
    lnw = jnp.log(w_ref[...])
    lnw_ref[...] = lnw
    lnp_ref[...] = jnp.broadcast_to(pow_ref[0, 0] * lnw, (lnw.shape[0], 16))


def _tc_embed(efe, sfe, ew, wea, bea, wsa, wsb, bsa, pow1):
    grid = (NPAD // BE,)
    return pl.pallas_call(
        _embed_body,
        grid=grid,
        in_specs=[
            pl.BlockSpec((BE, 128), lambda i: (i, 0)),
            pl.BlockSpec((BE, 128), lambda i: (i, 0)),
            pl.BlockSpec((BE, 1), lambda i: (i, 0)),
            pl.BlockSpec((128, 32), lambda i: (0, 0)),
            pl.BlockSpec((1, 32), lambda i: (0, 0)),
            pl.BlockSpec((128, 32), lambda i: (0, 0)),
            pl.BlockSpec((1, 32), lambda i: (0, 0)),
            pl.BlockSpec((1, 32), lambda i: (0, 0)),
            pl.BlockSpec(memory_space=pltpu.SMEM),
        ],
        out_specs=[
            pl.BlockSpec((BE, 64), lambda i: (i, 0)),
            pl.BlockSpec((BE, 16), lambda i: (i, 0)),
            pl.BlockSpec((BE, 1), lambda i: (i, 0)),
        ],
        out_shape=[jax.ShapeDtypeStruct((NPAD, 64), F32),
                   jax.ShapeDtypeStruct((NPAD, 16), F32),
                   jax.ShapeDtypeStruct((NPAD, 1), F32)],
    )(efe, sfe, ew, wea, bea, wsa, wsb, bsa, pow1)


def _edge_body(p_ref, l_ref, a_ref, c_ref, b1_ref, w2g_ref, b2g_ref,
               w2m_ref, b2m_ref, zab_ref):
    i = pl.program_id(0)
    bf16 = jnp.bfloat16
    pt = p_ref[...]
    sfe = pt[:, :64].astype(bf16)
    bf = pt[:, 64:].astype(bf16)
    row8 = lax.broadcasted_iota(jnp.int32, (BE, BE // 8), 0) // 8
    col8 = lax.broadcasted_iota(jnp.int32, (BE, BE // 8), 1)
    asel = jnp.where(row8 == col8, 1.0, 0.0)
    rep = jnp.dot(asel, l_ref[...], preferred_element_type=F32)
    lane = lax.broadcasted_iota(jnp.int32, (BE, 128), 1)
    rmod = (lax.broadcasted_iota(jnp.int32, (BE, 128), 0) % 8) * 16
    lnpw = jnp.sum(jnp.where(lane == rmod, rep, 0.0), axis=1, keepdims=True)
    h = (jnp.dot(sfe, a_ref[...].astype(bf16), preferred_element_type=F32)
         + jnp.dot(bf, c_ref[...].astype(bf16),
                   preferred_element_type=F32) + b1_ref[...])
    h = jnp.where(h > 0, h, 0.01 * h)
    hg = h[:, :256]
    hm = h[:, 256:]
    g = jnp.sum(hg * w2g_ref[...], axis=1, keepdims=True) + b2g_ref[0, 0]
    eid = i * BE + lax.broadcasted_iota(jnp.int32, (BE, 1), 0)
    u = jnp.where(eid < _E, jnp.exp(g + lnpw), 0.0)
    msg = jnp.dot(hm.astype(bf16), w2m_ref[...].astype(bf16),
                  preferred_element_type=F32) + b2m_ref[...]
    z = msg * u
    zab_ref[...] = jnp.concatenate(
        [z[:, :32], u, jnp.zeros((BE, 15), F32), z[:, 32:],
         jnp.zeros((BE, 48), F32)], axis=1)


def _tc_edge(pg, lg, wp):
    grid = (EPAD // BE,)
    return pl.pallas_call(
        _edge_body,
        grid=grid,
        in_specs=[
            pl.BlockSpec((BE, 128), lambda i: (i, 0)),
            pl.BlockSpec((BE // 8, 128), lambda i: (i, 0)),
            pl.BlockSpec((64, 512), lambda i: (0, 0)),
            pl.BlockSpec((64, 512), lambda i: (0, 0)),
            pl.BlockSpec((1, 512), lambda i: (0, 0)),
            pl.BlockSpec((1, 256), lambda i: (0, 0)),
            pl.BlockSpec(memory_space=pltpu.SMEM),
            pl.BlockSpec((256, 64), lambda i: (0, 0)),
            pl.BlockSpec((1, 64), lambda i: (0, 0)),
        ],
        out_specs=pl.BlockSpec((BE, 128), lambda i: (i, 0)),
        out_shape=jax.ShapeDtypeStruct((EPAD, 128), F32),
    )(pg, lg, wp["a"], wp["c"], wp["b1"], wp["w2g"], wp["b2g"], wp["w2m"],
      wp["b2m"])


def _finalize_body(sa_ref, sb_ref, fp_ref, lnw_ref, pow_ref,
                   fea_ref, lnp_ref):
    sa = sa_ref[...]
    sb = sb_ref[...]
    z = jnp.concatenate([sa[:, :32], sb[:, :32]], axis=1)
    u = sa[:, 32:33]
    fea_ref[...] = z / (u + 1e-10) + fp_ref[...]
    lnpw = pow_ref[0, 0] * lnw_ref[...]
    lnp_ref[...] = jnp.broadcast_to(lnpw, (lnpw.shape[0], 16))


def _tc_finalize(sa, sb, fprev, lnw, pownext):
    grid = (NPAD // BE,)
    return pl.pallas_call(
        _finalize_body,
        grid=grid,
        in_specs=[
            pl.BlockSpec((BE, W), lambda i: (i, 0)),
            pl.BlockSpec((BE, W), lambda i: (i, 0)),
            pl.BlockSpec((BE, 64), lambda i: (i, 0)),
            pl.BlockSpec((BE, 1), lambda i: (i, 0)),
            pl.BlockSpec(memory_space=pltpu.SMEM),
        ],
        out_specs=[pl.BlockSpec((BE, 64), lambda i: (i, 0)),
                   pl.BlockSpec((BE, 16), lambda i: (i, 0))],
        out_shape=[jax.ShapeDtypeStruct((NPAD, 64), F32),
                   jax.ShapeDtypeStruct((NPAD, 16), F32)],
    )(sa, sb, fprev, lnw, pownext)


def _cry_body(f_ref, lnw_ref, a_ref, b1_ref, w2g_ref, b2g_ref, w2m_ref,
              b2m_ref, pow_ref, zab_ref):
    i = pl.program_id(0)
    bf16 = jnp.bfloat16
    x = f_ref[...]
    h = jnp.dot(x.astype(bf16), a_ref[...].astype(bf16),
                preferred_element_type=F32) + b1_ref[...]
    h = jnp.where(h > 0, h, 0.01 * h)
    hg = h[:, :256]
    hm = h[:, 256:]
    g = jnp.sum(hg * w2g_ref[...], axis=1, keepdims=True) + b2g_ref[0, 0]
    nid = i * BE + lax.broadcasted_iota(jnp.int32, (BE, 1), 0)
    u = jnp.where(nid < _N, jnp.exp(g + pow_ref[0, 0] * lnw_ref[...]), 0.0)
    msg = jnp.dot(hm.astype(bf16), w2m_ref[...].astype(bf16),
                  preferred_element_type=F32) + b2m_ref[...]
    z = msg * u
    zab_ref[...] = jnp.concatenate(
        [z[:, :32], u, jnp.zeros((BE, 15), F32), z[:, 32:],
         jnp.zeros((BE, 48), F32)], axis=1)


def _tc_cry(fea, lnw, wp):
    grid = (NPAD // BE,)
    return pl.pallas_call(
        _cry_body,
        grid=grid,
        in_specs=[
            pl.BlockSpec((BE, 64), lambda i: (i, 0)),
            pl.BlockSpec((BE, 1), lambda i: (i, 0)),
            pl.BlockSpec((64, 512), lambda i: (0, 0)),
            pl.BlockSpec((1, 512), lambda i: (0, 0)),
            pl.BlockSpec((1, 256), lambda i: (0, 0)),
            pl.BlockSpec(memory_space=pltpu.SMEM),
            pl.BlockSpec((256, 64), lambda i: (0, 0)),
            pl.BlockSpec((1, 64), lambda i: (0, 0)),
            pl.BlockSpec(memory_space=pltpu.SMEM),
        ],
        out_specs=pl.BlockSpec((BE, 128), lambda i: (i, 0)),
        out_shape=jax.ShapeDtypeStruct((NPAD, 128), F32),
    )(fea, lnw, wp["a"], wp["b1"], wp["w2g"], wp["b2g"], wp["w2m"],
      wp["b2m"], wp["pow"])


def _cryfin_body(sa_ref, sb_ref, zab_ref):
    i = pl.program_id(0)
    sa = sa_ref[...]
    sb = sb_ref[...]
    z = jnp.concatenate([sa[:, :32], sb[:, :32]], axis=1)
    u = sa[:, 32:33]
    cry = z / (u + 1e-10)
    rid = i * BE + lax.broadcasted_iota(jnp.int32, (BE, 1), 0)
    ones = jnp.where(rid < _C, 1.0, 0.0)
    zab_ref[...] = jnp.concatenate(
        [cry[:, :32], ones, jnp.zeros((BE, 15), F32), cry[:, 32:],
         jnp.zeros((BE, 48), F32)], axis=1)


def _tc_cryfin(sa, sb):
    grid = (CPAD // BE,)
    return pl.pallas_call(
        _cryfin_body,
        grid=grid,
        in_specs=[pl.BlockSpec((BE, W), lambda i: (i, 0)),
                  pl.BlockSpec((BE, W), lambda i: (i, 0))],
        out_specs=pl.BlockSpec((BE, 128), lambda i: (i, 0)),
        out_shape=jax.ShapeDtypeStruct((CPAD, 128), F32),
    )(sa, sb)


def _final_body(sa_ref, sb_ref, out_ref):
    sa = sa_ref[...]
    sb = sb_ref[...]
    sums = jnp.concatenate([sa[:, :32], sb[:, :32]], axis=1)
    counts = sa[:, 32:33]
    out_ref[...] = sums / jnp.maximum(counts, 1.0)


def _tc_final(sa, sb):
    grid = (APAD // BE,)
    return pl.pallas_call(
        _final_body,
        grid=grid,
        in_specs=[pl.BlockSpec((BE, W), lambda i: (i, 0)),
                  pl.BlockSpec((BE, W), lambda i: (i, 0))],
        out_specs=pl.BlockSpec((BE, 64), lambda i: (i, 0)),
        out_shape=jax.ShapeDtypeStruct((APAD, 64), F32),
    )(sa, sb)


def _pack_wap(p, din):
    wg1, bg1 = p["gate"]["fcs"][0]
    wg2, bg2 = p["gate"]["out"]
    wm1, bm1 = p["msg"]["fcs"][0]
    wm2, bm2 = p["msg"]["out"]
    d = {}
    if din == 128:
        d["a"] = jnp.concatenate([wg1[:, :64].T, wm1[:, :64].T], axis=1)
        d["c"] = jnp.concatenate([wg1[:, 64:].T, wm1[:, 64:].T], axis=1)
    else:
        d["a"] = jnp.concatenate([wg1.T, wm1.T], axis=1)
    d["b1"] = jnp.concatenate([bg1, bm1]).reshape(1, 512)
    d["w2g"] = wg2.reshape(1, 256)
    d["b2g"] = bg2.reshape(1, 1)
    d["w2m"] = wm2.T
    d["b2m"] = bm2.reshape(1, 64)
    d["pow"] = p["pow"].reshape(1, 1)
    return d


def kernel(elem_weights, elem_fea, sym_fea, self_fea_idx, nbr_fea_idx,
           cry_elem_idx, aug_cry_idx, params):
    ew = jnp.pad(elem_weights, ((0, NPAD - _N), (0, 0)), constant_values=1.0)
    efe = jnp.pad(elem_fea, ((0, NPAD - _N), (0, 0)))
    sfe = jnp.pad(sym_fea, ((0, NPAD - _N), (0, 0)))
    sidx = jnp.pad(self_fea_idx, (0, EPAD - _E)).reshape(EPAD // 128, 128)
    nidx = jnp.pad(nbr_fea_idx, (0, EPAD - _E)).reshape(EPAD // 128, 128)
    sidx64 = sidx.reshape(EPAD // 64, 64)
    cidx = jnp.pad(cry_elem_idx, (0, NPAD - _N)).reshape(NPAD // 64, 64)
    aidx = jnp.pad(aug_cry_idx, (0, CPAD - _C)).reshape(CPAD // 128, 128)

    graph_w = [_pack_wap(gp["heads"][0], 128) for gp in params["graphs"]]
    cry_w = _pack_wap(params["cry_pool"][0], 64)

    wemb, bemb = params["elem_embed"]
    wsym, bsym = params["sym_embed"]
    fea, lnp, lnw = _tc_embed(
        efe, sfe, ew, wemb.T, bemb.reshape(1, 32), wsym[:, :128].T,
        wsym[:, 128:].T, bsym.reshape(1, 32), graph_w[0]["pow"])

    for layer in range(3):
        pg, lg = _sc_gather(fea, lnp, sidx, nidx)
        zab = _tc_edge(pg, lg.reshape(EPAD // 8, 128), graph_w[layer])
        sa, sb = _sc_segsum(zab, sidx64, EPAD, NPAD, 64)
        pnext = (graph_w[layer + 1]["pow"] if layer < 2
                 else graph_w[layer]["pow"])
        fea, lnp = _tc_finalize(sa, sb, fea, lnw, pnext)

    zab = _tc_cry(fea, lnw, cry_w)
    sa, sb = _sc_segsum(zab, cidx, NPAD, CPAD, 64)
    zab2 = _tc_cryfin(sa, sb)
    sa2, sb2 = _sc_segsum(zab2, aidx, CPAD, APAD, 128)
    out = _tc_final(sa2, sb2)
    return out[:_A]
